# baseline (speedup 1.0000x reference)
import functools

import jax
import jax.numpy as jnp
from jax import lax
from jax.experimental import pallas as pl
from jax.experimental.pallas import tpu as pltpu
from jax.experimental.pallas import tpu_sc as plsc

_NTOK = 1000000
_S = 8192
_H = 64
_B = 4096

_NC = 2
_NS = 16
_NW = _NC * _NS

_BB = 512
_SWP = _S + 8


def _sc_bias_gather(bias, labels, sample_ids):
    s_per = _S // _NW
    t_per = _B // _NW
    mesh = plsc.VectorSubcoreMesh(core_axis_name="c", subcore_axis_name="s")

    @functools.partial(
        pl.kernel,
        out_type=[
            jax.ShapeDtypeStruct((_S,), jnp.float32),
            jax.ShapeDtypeStruct((_B,), jnp.float32),
        ],
        mesh=mesh,
        scratch_types=[
            pltpu.VMEM((s_per,), jnp.int32),
            pltpu.VMEM((s_per,), jnp.float32),
            pltpu.VMEM((t_per,), jnp.int32),
            pltpu.VMEM((t_per,), jnp.float32),
            pltpu.SemaphoreType.DMA,
        ],
    )
    def gather_kernel(bias_hbm, labels_hbm, sids_hbm, sb_hbm, tb_hbm,
                      sidx_v, sbias_v, tidx_v, tbias_v, sem_b):
        wid = lax.axis_index("s") * _NC + lax.axis_index("c")
        sbase = wid * s_per
        tbase = wid * t_per
        pltpu.sync_copy(sids_hbm.at[pl.ds(sbase, s_per)], sidx_v)
        pltpu.sync_copy(labels_hbm.at[pl.ds(tbase, t_per)], tidx_v)
        pltpu.make_async_copy(bias_hbm.at[sidx_v], sbias_v, sem_b).start()
        pltpu.make_async_copy(bias_hbm.at[tidx_v], tbias_v, sem_b).start()
        pltpu.make_async_copy(bias_hbm.at[pl.ds(0, s_per)], sbias_v, sem_b).wait()
        pltpu.make_async_copy(bias_hbm.at[pl.ds(0, t_per)], tbias_v, sem_b).wait()
        pltpu.sync_copy(sbias_v, sb_hbm.at[pl.ds(sbase, s_per)])
        pltpu.sync_copy(tbias_v, tb_hbm.at[pl.ds(tbase, t_per)])

    return gather_kernel(bias, labels, sample_ids)


def _tc_body(sids_sm, labels_sm, x_ref, tf_ref, sf_ref, sb_ref, tb_ref,
             w_any, out_ref, swp_scr, tw_scr, sem_s, sem_l):
    i = pl.program_id(0)

    @pl.when(i == 0)
    def _():
        def fire_s(j, carry):
            pltpu.make_async_copy(
                w_any.at[pl.ds(sids_sm[j], 1), :],
                swp_scr.at[pl.ds(j + 1, 1), :], sem_s).start()
            return carry
        lax.fori_loop(0, _S, fire_s, 0)

    def fire_t(j, carry):
        pltpu.make_async_copy(
            w_any.at[pl.ds(labels_sm[i * _BB + j], 1), :],
            tw_scr.at[pl.ds(j, 1), :], sem_l).start()
        return carry
    lax.fori_loop(0, _BB, fire_t, 0)

    pltpu.make_async_copy(w_any.at[pl.ds(0, _BB), :], tw_scr, sem_l).wait()

    @pl.when(i == 0)
    def _():
        pltpu.make_async_copy(w_any.at[pl.ds(0, _S), :],
                              swp_scr.at[pl.ds(1, _S), :], sem_s).wait()

    x = x_ref[...]
    tl = (jnp.sum(x * tw_scr[...], axis=1) + tb_ref[...]
          - jnp.log(tf_ref[...]))
    mm = lax.dot_general(x, swp_scr[...], (((1,), (1,)), ((), ())),
                         preferred_element_type=jnp.float32)[:, :_S + 1]
    sbp = jnp.concatenate(
        [jnp.zeros((1,), jnp.float32),
         sb_ref[...] - jnp.log(sf_ref[...])])
    col = lax.broadcasted_iota(jnp.int32, (_BB, _S + 1), 1)
    out_ref[...] = jnp.where(col == 0, tl[:, None], mm + sbp[None, :])


def _tc_logits(sample_ids, labels, inputs, true_freq, sample_freq, sb, tb,
               weight):
    grid_spec = pltpu.PrefetchScalarGridSpec(
        num_scalar_prefetch=2,
        grid=(_B // _BB,),
        in_specs=[
            pl.BlockSpec((_BB, _H), lambda i, *_: (i, 0)),
            pl.BlockSpec((_BB,), lambda i, *_: (i,)),
            pl.BlockSpec((_S,), lambda i, *_: (0,)),
            pl.BlockSpec((_S,), lambda i, *_: (0,)),
            pl.BlockSpec((_BB,), lambda i, *_: (i,)),
            pl.BlockSpec(memory_space=pl.ANY),
        ],
        out_specs=pl.BlockSpec((_BB, _S + 1), lambda i, *_: (i, 0)),
        scratch_shapes=[
            pltpu.VMEM((_SWP, _H), jnp.float32),
            pltpu.VMEM((_BB, _H), jnp.float32),
            pltpu.SemaphoreType.DMA,
            pltpu.SemaphoreType.DMA,
        ],
    )
    return pl.pallas_call(
        _tc_body,
        grid_spec=grid_spec,
        out_shape=jax.ShapeDtypeStruct((_B, _S + 1), jnp.float32),
    )(sample_ids, labels, inputs, true_freq, sample_freq, sb, tb, weight)


def kernel(inputs, labels, sample_ids, true_freq, sample_freq, weight, bias):
    sb, tb = _sc_bias_gather(bias, labels, sample_ids)
    logits = _tc_logits(sample_ids, labels, inputs, true_freq, sample_freq,
                        sb, tb, weight)
    new_targets = jnp.zeros((_B,), dtype=jnp.int32)
    return (logits, new_targets)


# speedup vs baseline: 1.1027x; 1.1027x over previous
import functools

import jax
import jax.numpy as jnp
from jax import lax
from jax.experimental import pallas as pl
from jax.experimental.pallas import tpu as pltpu
from jax.experimental.pallas import tpu_sc as plsc

_NTOK = 1000000
_S = 8192
_H = 64
_B = 4096

_NC = 2
_NS = 16
_NW = _NC * _NS


def _sc_gather(weight, bias, labels, sample_ids):
    s_per = _S // _NW
    t_per = _B // _NW
    mesh = plsc.VectorSubcoreMesh(core_axis_name="c", subcore_axis_name="s")

    @functools.partial(
        pl.kernel,
        out_type=[
            jax.ShapeDtypeStruct((_S, _H), jnp.float32),
            jax.ShapeDtypeStruct((_S,), jnp.float32),
            jax.ShapeDtypeStruct((_B, _H), jnp.float32),
            jax.ShapeDtypeStruct((_B,), jnp.float32),
        ],
        mesh=mesh,
        compiler_params=pltpu.CompilerParams(needs_layout_passes=True),
        scratch_types=[
            pltpu.VMEM((s_per,), jnp.int32),
            pltpu.VMEM((s_per, _H), jnp.float32),
            pltpu.VMEM((s_per,), jnp.float32),
            pltpu.VMEM((t_per,), jnp.int32),
            pltpu.VMEM((t_per, _H), jnp.float32),
            pltpu.VMEM((t_per,), jnp.float32),
            pltpu.SemaphoreType.DMA,
            pltpu.SemaphoreType.DMA,
        ],
    )
    def gather_kernel(weight_hbm, bias_hbm, labels_hbm, sids_hbm,
                      sw_hbm, sb_hbm, tw_hbm, tb_hbm,
                      sidx_v, srows_v, sbias_v, tidx_v, trows_v, tbias_v,
                      sem_w, sem_b):
        wid = lax.axis_index("s") * _NC + lax.axis_index("c")
        sbase = wid * s_per
        tbase = wid * t_per
        pltpu.sync_copy(sids_hbm.at[pl.ds(sbase, s_per)], sidx_v)
        pltpu.sync_copy(labels_hbm.at[pl.ds(tbase, t_per)], tidx_v)
        pltpu.make_async_copy(bias_hbm.at[sidx_v], sbias_v, sem_b).start()
        pltpu.make_async_copy(bias_hbm.at[tidx_v], tbias_v, sem_b).start()

        def make_fire(idx_ref, rows_ref):
            def fire(c, carry):
                vec = idx_ref[pl.ds(c * 16, 16)]
                for k in range(16):
                    idx = vec[k]
                    pltpu.make_async_copy(
                        weight_hbm.at[pl.ds(idx, 1), :],
                        rows_ref.at[pl.ds(c * 16 + k, 1), :], sem_w).start()
                return carry
            return fire

        lax.fori_loop(0, s_per // 16, make_fire(sidx_v, srows_v), 0)
        lax.fori_loop(0, t_per // 16, make_fire(tidx_v, trows_v), 0)
        pltpu.make_async_copy(weight_hbm.at[pl.ds(0, s_per), :],
                              srows_v, sem_w).wait()
        pltpu.make_async_copy(weight_hbm.at[pl.ds(0, t_per), :],
                              trows_v, sem_w).wait()
        pltpu.make_async_copy(bias_hbm.at[pl.ds(0, s_per)], sbias_v, sem_b).wait()
        pltpu.make_async_copy(bias_hbm.at[pl.ds(0, t_per)], tbias_v, sem_b).wait()
        pltpu.sync_copy(srows_v, sw_hbm.at[pl.ds(sbase, s_per)])
        pltpu.sync_copy(sbias_v, sb_hbm.at[pl.ds(sbase, s_per)])
        pltpu.sync_copy(trows_v, tw_hbm.at[pl.ds(tbase, t_per)])
        pltpu.sync_copy(tbias_v, tb_hbm.at[pl.ds(tbase, t_per)])

    return gather_kernel(weight, bias, labels, sample_ids)


_BB = 512


def _tc_body(in_ref, tw_ref, tb_ref, tf_ref, sw_ref, sb_ref, sf_ref, out_ref):
    x = in_ref[...]
    tl = (jnp.sum(x * tw_ref[...], axis=1) + tb_ref[...]
          - jnp.log(tf_ref[...]))
    mm = lax.dot_general(x, sw_ref[...], (((1,), (1,)), ((), ())),
                         preferred_element_type=jnp.float32)
    sl = mm + (sb_ref[...] - jnp.log(sf_ref[...]))[None, :]
    out_ref[...] = jnp.concatenate([tl[:, None], sl], axis=1)


def _tc_logits(inputs, tw, tb, true_freq, sw, sb, sample_freq, interpret=False):
    grid = (_B // _BB,)
    return pl.pallas_call(
        _tc_body,
        grid=grid,
        in_specs=[
            pl.BlockSpec((_BB, _H), lambda i: (i, 0)),
            pl.BlockSpec((_BB, _H), lambda i: (i, 0)),
            pl.BlockSpec((_BB,), lambda i: (i,)),
            pl.BlockSpec((_BB,), lambda i: (i,)),
            pl.BlockSpec((_S, _H), lambda i: (0, 0)),
            pl.BlockSpec((_S,), lambda i: (0,)),
            pl.BlockSpec((_S,), lambda i: (0,)),
        ],
        out_specs=pl.BlockSpec((_BB, _S + 1), lambda i: (i, 0)),
        out_shape=jax.ShapeDtypeStruct((_B, _S + 1), jnp.float32),
        interpret=interpret,
    )(inputs, tw, tb, true_freq, sw, sb, sample_freq)


def kernel(inputs, labels, sample_ids, true_freq, sample_freq, weight, bias):
    sw, sb, tw, tb = _sc_gather(weight, bias, labels, sample_ids)
    logits = _tc_logits(inputs, tw, tb, true_freq, sw, sb, sample_freq)
    new_targets = jnp.zeros((_B,), dtype=jnp.int32)
    return (logits, new_targets)


# speedup vs baseline: 1.2292x; 1.1148x over previous
import functools

import jax
import jax.numpy as jnp
from jax import lax
from jax.experimental import pallas as pl
from jax.experimental.pallas import tpu as pltpu
from jax.experimental.pallas import tpu_sc as plsc

_NTOK = 1000000
_S = 8192
_H = 64
_B = 4096

_NC = 2
_NS = 16
_NW = _NC * _NS

_BB = 512
_SWP = _S + 8


def _sc_bias_gather(bias, labels, sample_ids):
    s_per = _S // _NW
    t_per = _B // _NW
    mesh = plsc.VectorSubcoreMesh(core_axis_name="c", subcore_axis_name="s")

    @functools.partial(
        pl.kernel,
        out_type=[
            jax.ShapeDtypeStruct((_S,), jnp.float32),
            jax.ShapeDtypeStruct((_B,), jnp.float32),
        ],
        mesh=mesh,
        scratch_types=[
            pltpu.VMEM((s_per,), jnp.int32),
            pltpu.VMEM((s_per,), jnp.float32),
            pltpu.VMEM((t_per,), jnp.int32),
            pltpu.VMEM((t_per,), jnp.float32),
            pltpu.SemaphoreType.DMA,
        ],
    )
    def gather_kernel(bias_hbm, labels_hbm, sids_hbm, sb_hbm, tb_hbm,
                      sidx_v, sbias_v, tidx_v, tbias_v, sem_b):
        wid = lax.axis_index("s") * _NC + lax.axis_index("c")
        sbase = wid * s_per
        tbase = wid * t_per
        pltpu.sync_copy(sids_hbm.at[pl.ds(sbase, s_per)], sidx_v)
        pltpu.sync_copy(labels_hbm.at[pl.ds(tbase, t_per)], tidx_v)
        pltpu.make_async_copy(bias_hbm.at[sidx_v], sbias_v, sem_b).start()
        pltpu.make_async_copy(bias_hbm.at[tidx_v], tbias_v, sem_b).start()
        pltpu.make_async_copy(bias_hbm.at[pl.ds(0, s_per)], sbias_v, sem_b).wait()
        pltpu.make_async_copy(bias_hbm.at[pl.ds(0, t_per)], tbias_v, sem_b).wait()
        pltpu.sync_copy(sbias_v, sb_hbm.at[pl.ds(sbase, s_per)])
        pltpu.sync_copy(tbias_v, tb_hbm.at[pl.ds(tbase, t_per)])

    return gather_kernel(bias, labels, sample_ids)


def _tc_body(sids_sm, labels_sm, xt_ref, tf_ref, tb_ref, sbt_ref, sft_ref,
             w_any, out_ref, sw_scr, tw_scr, sem_s, sem_l):
    i = pl.program_id(0)

    @pl.when(i == 0)
    def _():
        def fire_s(j, carry):
            pltpu.make_async_copy(
                w_any.at[pl.ds(sids_sm[j], 1), :],
                sw_scr.at[pl.ds(j + 1, 1), :], sem_s).start()
            return carry
        lax.fori_loop(0, _S, fire_s, 0)

    def fire_t(j, carry):
        pltpu.make_async_copy(
            w_any.at[pl.ds(labels_sm[i * _BB + j], 1), :],
            tw_scr.at[pl.ds(j, 1), :], sem_l).start()
        return carry
    lax.fori_loop(0, _BB, fire_t, 0)

    pltpu.make_async_copy(w_any.at[pl.ds(0, _BB), :], tw_scr, sem_l).wait()

    @pl.when(i == 0)
    def _():
        pltpu.make_async_copy(w_any.at[pl.ds(0, _S), :],
                              sw_scr.at[pl.ds(1, _S), :], sem_s).wait()

    xt = xt_ref[...]
    twt = tw_scr[...].T
    tl = (jnp.sum(twt * xt, axis=0) + tb_ref[...]
          - jnp.log(tf_ref[...]))
    mm = lax.dot_general(sw_scr[...], xt, (((1,), (0,)), ((), ())),
                         preferred_element_type=jnp.float32)[:_S + 1]
    cb = jnp.concatenate(
        [jnp.zeros((1, 1), jnp.float32),
         sbt_ref[...] - jnp.log(sft_ref[...])], axis=0)
    row = lax.broadcasted_iota(jnp.int32, (_S + 1, _BB), 0)
    out_ref[...] = jnp.where(row == 0, tl[None, :], mm + cb)


def _tc_logits_t(sample_ids, labels, inputs_t, true_freq, sbt, tb, sft,
                 weight):
    grid_spec = pltpu.PrefetchScalarGridSpec(
        num_scalar_prefetch=2,
        grid=(_B // _BB,),
        in_specs=[
            pl.BlockSpec((_H, _BB), lambda i, *_: (0, i)),
            pl.BlockSpec((_BB,), lambda i, *_: (i,)),
            pl.BlockSpec((_BB,), lambda i, *_: (i,)),
            pl.BlockSpec((_S, 1), lambda i, *_: (0, 0)),
            pl.BlockSpec((_S, 1), lambda i, *_: (0, 0)),
            pl.BlockSpec(memory_space=pl.ANY),
        ],
        out_specs=pl.BlockSpec((_S + 1, _BB), lambda i, *_: (0, i)),
        scratch_shapes=[
            pltpu.VMEM((_SWP, _H), jnp.float32),
            pltpu.VMEM((_BB, _H), jnp.float32),
            pltpu.SemaphoreType.DMA,
            pltpu.SemaphoreType.DMA,
        ],
    )
    return pl.pallas_call(
        _tc_body,
        grid_spec=grid_spec,
        out_shape=jax.ShapeDtypeStruct((_S + 1, _B), jnp.float32),
    )(sample_ids, labels, inputs_t, true_freq, tb, sbt, sft, weight)


def kernel(inputs, labels, sample_ids, true_freq, sample_freq, weight, bias):
    sb, tb = _sc_bias_gather(bias, labels, sample_ids)
    logits_t = _tc_logits_t(sample_ids, labels, inputs.T, true_freq,
                            sb[:, None], tb, sample_freq[:, None], weight)
    new_targets = jnp.zeros((_B,), dtype=jnp.int32)
    return (logits_t.T, new_targets)
